# baseline (speedup 1.0000x reference)
import jax
import jax.numpy as jnp
from jax import lax
from jax.experimental import pallas as pl
from jax.experimental.pallas import tpu as pltpu

BB = 8
UNROLL = 8
CD = 256


def _pool_kernel(w_hbm, x_s, x_v, out_ref, wtab, sem):
    e = pl.program_id(0)
    L = x_s.shape[1]

    @pl.when(pl.program_id(1) == 0)
    def _load_table():
        cp = pltpu.make_async_copy(w_hbm.at[e], wtab, sem)
        cp.start()
        cp.wait()

    mask = jnp.int32(-65536)
    rows = []
    for r in range(BB):
        def body(c, accs, r=r):
            aa, ab = accs
            base = c * UNROLL
            for j in range(UNROLL):
                idx = x_s[r, base + j]
                v = wtab[idx]
                aa = aa + pltpu.bitcast(v & mask, jnp.float32)
                ab = ab + pltpu.bitcast(v << 16, jnp.float32)
            return (aa, ab)

        z = jnp.zeros((1, 128), jnp.float32)
        aa, ab = lax.fori_loop(0, L // UNROLL, body, (z, z))
        rows.append(jnp.concatenate([aa, ab], axis=1))
    blk = jnp.concatenate(rows, axis=0)

    pos = lax.broadcasted_iota(jnp.int32, (BB, L), 1)
    lens = jnp.min(jnp.where(x_v[...] == 0, pos, L), axis=1, keepdims=True)
    inv = 1.0 / lens.astype(jnp.float32)
    out_ref[:, 0, 0, :] = blk * inv


def _mlp_kernel(y_ref, w1t_ref, b1_ref, w2t_ref, b2_ref, out_ref):
    h = jnp.dot(y_ref[...], w1t_ref[...], preferred_element_type=jnp.float32)
    h = jnp.maximum(h + b1_ref[...], 0.0)
    out_ref[...] = (
        jnp.dot(h, w2t_ref[...], preferred_element_type=jnp.float32)
        + b2_ref[...]
    )


@jax.jit
def _run(x, weight, w1, b1, w2, b2):
    V, E = weight.shape
    B, L = x.shape
    H = w1.shape[0]
    O = w2.shape[0]
    EP = 2 * CD

    wpad = jnp.pad(weight, ((0, 0), (0, EP - E)))
    u32 = lax.bitcast_convert_type(
        wpad.astype(jnp.bfloat16), jnp.uint16
    ).astype(jnp.uint32)
    pk = jnp.stack(
        [
            (u32[:, e * CD : e * CD + 128] << 16) | u32[:, e * CD + 128 : (e + 1) * CD]
            for e in range(2)
        ],
        axis=0,
    )
    pk = lax.bitcast_convert_type(pk, jnp.int32).reshape(2, V, 1, 128)

    pooled = pl.pallas_call(
        _pool_kernel,
        grid=(2, B // BB),
        in_specs=[
            pl.BlockSpec(memory_space=pl.ANY),
            pl.BlockSpec((BB, L), lambda e, b: (b, 0), memory_space=pltpu.SMEM),
            pl.BlockSpec((BB, L), lambda e, b: (b, 0)),
        ],
        out_specs=pl.BlockSpec((BB, 1, 1, CD), lambda e, b: (b, e, 0, 0)),
        out_shape=jax.ShapeDtypeStruct((B, 2, 1, CD), jnp.float32),
        scratch_shapes=[
            pltpu.VMEM((V, 1, 128), jnp.int32),
            pltpu.SemaphoreType.DMA,
        ],
        compiler_params=pltpu.CompilerParams(
            dimension_semantics=("parallel", "arbitrary"),
        ),
        name="embed_pool",
    )(pk, x, x)
    y = pooled.reshape(B, EP)

    OP = 128
    w1t = jnp.pad(w1, ((0, 0), (0, EP - E))).T
    w2t = jnp.pad(w2, ((0, OP - O), (0, 0))).T
    b1r = b1.reshape(1, H)
    b2r = jnp.pad(b2, (0, OP - O)).reshape(1, OP)
    MB = min(256, B)
    out = pl.pallas_call(
        _mlp_kernel,
        grid=(B // MB,),
        in_specs=[
            pl.BlockSpec((MB, EP), lambda b: (b, 0)),
            pl.BlockSpec((EP, H), lambda b: (0, 0)),
            pl.BlockSpec((1, H), lambda b: (0, 0)),
            pl.BlockSpec((H, OP), lambda b: (0, 0)),
            pl.BlockSpec((1, OP), lambda b: (0, 0)),
        ],
        out_specs=pl.BlockSpec((MB, OP), lambda b: (b, 0)),
        out_shape=jax.ShapeDtypeStruct((B, OP), jnp.float32),
        compiler_params=pltpu.CompilerParams(dimension_semantics=("parallel",)),
        name="pool_mlp",
    )(y, w1t, b1r, w2t, b2r)
    return out[:, :O]


def kernel(x, weight, w1, b1, w2, b2):
    return _run(x, weight, w1, b1, w2, b2)


# speedup vs baseline: 1.9127x; 1.9127x over previous
import jax
import jax.numpy as jnp
from jax import lax
from jax.experimental import pallas as pl
from jax.experimental.pallas import tpu as pltpu

BB = 8
UNROLL = 8
CD = 256


def _pool_kernel(w_hbm, x_s, lens_s, x_v, out_ref, wtab, sem):
    e = pl.program_id(0)
    L = x_s.shape[1]

    @pl.when(pl.program_id(1) == 0)
    def _load_table():
        cp = pltpu.make_async_copy(w_hbm.at[e], wtab, sem)
        cp.start()
        cp.wait()

    mask = jnp.int32(-65536)
    v0 = wtab[0]
    w0a = pltpu.bitcast(v0 & mask, jnp.float32)
    w0b = pltpu.bitcast(v0 << 16, jnp.float32)

    rows = [None] * BB
    for r in range(0, BB, 2):
        n = jnp.maximum(lens_s[0, 0, r], lens_s[0, 0, r + 1])
        nc = (n + (UNROLL - 1)) // UNROLL

        def body(c, accs, r=r):
            a0, b0, a1, b1 = accs
            base = c * UNROLL
            for j in range(UNROLL):
                i0 = x_s[r, base + j]
                i1 = x_s[r + 1, base + j]
                u = wtab[i0]
                v = wtab[i1]
                a0 = a0 + pltpu.bitcast(u & mask, jnp.float32)
                b0 = b0 + pltpu.bitcast(u << 16, jnp.float32)
                a1 = a1 + pltpu.bitcast(v & mask, jnp.float32)
                b1 = b1 + pltpu.bitcast(v << 16, jnp.float32)
            return (a0, b0, a1, b1)

        z = jnp.zeros((1, 128), jnp.float32)
        a0, b0, a1, b1 = lax.fori_loop(0, nc, body, (z, z, z, z))
        rem = (L - nc * UNROLL).astype(jnp.float32)
        rows[r] = jnp.concatenate([a0 + w0a * rem, b0 + w0b * rem], axis=1)
        rows[r + 1] = jnp.concatenate([a1 + w0a * rem, b1 + w0b * rem], axis=1)
    blk = jnp.concatenate(rows, axis=0)

    pos = lax.broadcasted_iota(jnp.int32, (BB, L), 1)
    lens = jnp.min(jnp.where(x_v[...] == 0, pos, L), axis=1, keepdims=True)
    inv = 1.0 / lens.astype(jnp.float32)
    out_ref[:, 0, 0, :] = blk * inv


def _mlp_kernel(y_ref, w1t_ref, b1_ref, w2t_ref, b2_ref, out_ref):
    h = jnp.dot(y_ref[...], w1t_ref[...], preferred_element_type=jnp.float32)
    h = jnp.maximum(h + b1_ref[...], 0.0)
    out_ref[...] = (
        jnp.dot(h, w2t_ref[...], preferred_element_type=jnp.float32)
        + b2_ref[...]
    )


@jax.jit
def _run(x, weight, w1, b1, w2, b2):
    V, E = weight.shape
    B, L = x.shape
    H = w1.shape[0]
    O = w2.shape[0]
    EP = 2 * CD

    wpad = jnp.pad(weight, ((0, 0), (0, EP - E)))
    u32 = lax.bitcast_convert_type(
        wpad.astype(jnp.bfloat16), jnp.uint16
    ).astype(jnp.uint32)
    pk = jnp.stack(
        [
            (u32[:, e * CD : e * CD + 128] << 16) | u32[:, e * CD + 128 : (e + 1) * CD]
            for e in range(2)
        ],
        axis=0,
    )
    pk = lax.bitcast_convert_type(pk, jnp.int32).reshape(2, V, 1, 128)

    posh = jnp.arange(L, dtype=jnp.int32)
    lens3 = (
        jnp.min(jnp.where(x == 0, posh[None, :], L), axis=1)
        .astype(jnp.int32)
        .reshape(B // BB, 1, BB)
    )

    pooled = pl.pallas_call(
        _pool_kernel,
        grid=(2, B // BB),
        in_specs=[
            pl.BlockSpec(memory_space=pl.ANY),
            pl.BlockSpec((BB, L), lambda e, b: (b, 0), memory_space=pltpu.SMEM),
            pl.BlockSpec((1, 1, BB), lambda e, b: (b, 0, 0), memory_space=pltpu.SMEM),
            pl.BlockSpec((BB, L), lambda e, b: (b, 0)),
        ],
        out_specs=pl.BlockSpec((BB, 1, 1, CD), lambda e, b: (b, e, 0, 0)),
        out_shape=jax.ShapeDtypeStruct((B, 2, 1, CD), jnp.float32),
        scratch_shapes=[
            pltpu.VMEM((V, 1, 128), jnp.int32),
            pltpu.SemaphoreType.DMA,
        ],
        compiler_params=pltpu.CompilerParams(
            dimension_semantics=("parallel", "arbitrary"),
            disable_bounds_checks=True,
        ),
        name="embed_pool",
    )(pk, x, lens3, x)
    y = pooled.reshape(B, EP)

    OP = 128
    w1t = jnp.pad(w1, ((0, 0), (0, EP - E))).T
    w2t = jnp.pad(w2, ((0, OP - O), (0, 0))).T
    b1r = b1.reshape(1, H)
    b2r = jnp.pad(b2, (0, OP - O)).reshape(1, OP)
    MB = min(256, B)
    out = pl.pallas_call(
        _mlp_kernel,
        grid=(B // MB,),
        in_specs=[
            pl.BlockSpec((MB, EP), lambda b: (b, 0)),
            pl.BlockSpec((EP, H), lambda b: (0, 0)),
            pl.BlockSpec((1, H), lambda b: (0, 0)),
            pl.BlockSpec((H, OP), lambda b: (0, 0)),
            pl.BlockSpec((1, OP), lambda b: (0, 0)),
        ],
        out_specs=pl.BlockSpec((MB, OP), lambda b: (b, 0)),
        out_shape=jax.ShapeDtypeStruct((B, OP), jnp.float32),
        compiler_params=pltpu.CompilerParams(
            dimension_semantics=("parallel",),
            disable_bounds_checks=True,
        ),
        name="pool_mlp",
    )(y, w1t, b1r, w2t, b2r)
    return out[:, :O]


def kernel(x, weight, w1, b1, w2, b2):
    return _run(x, weight, w1, b1, w2, b2)
